# baseline (speedup 1.0000x reference)
import functools

import jax
import jax.numpy as jnp
from jax import lax
from jax.experimental import pallas as pl
from jax.experimental.pallas import tpu as pltpu
from jax.experimental.pallas import tpu_sc as plsc

D = 128
H = 50
L = 16
NC = 2
NS = 16
NW = NC * NS
NCH = D // L
HP = 64


def _sc_gather_attend(items, users, history_users, emb_item, emb_user):
    B = items.shape[0]
    b_per_w = B // NW
    mesh = plsc.VectorSubcoreMesh(
        core_axis_name="c", subcore_axis_name="s",
        num_cores=NC, num_subcores=NS)

    @functools.partial(
        pl.kernel,
        mesh=mesh,
        compiler_params=pltpu.CompilerParams(needs_layout_passes=False),
        out_type=(
            jax.ShapeDtypeStruct((B, D), jnp.float32),
            jax.ShapeDtypeStruct((B, D), jnp.float32),
            jax.ShapeDtypeStruct((B, D), jnp.float32),
        ),
        scratch_types=[
            pltpu.VMEM((b_per_w,), jnp.int32),
            pltpu.VMEM((b_per_w,), jnp.int32),
            pltpu.VMEM((b_per_w, H), jnp.int32),
            pltpu.VMEM((b_per_w, D), jnp.float32),
            pltpu.VMEM((b_per_w, D), jnp.float32),
            pltpu.VMEM((H, D), jnp.float32),
            pltpu.VMEM((H, D), jnp.float32),
            pltpu.VMEM((H, D), jnp.float32),
            pltpu.VMEM((H, D), jnp.float32),
            pltpu.VMEM((b_per_w, D), jnp.float32),
            pltpu.VMEM((HP * (L + 1),), jnp.float32),
            pltpu.VMEM((HP + L,), jnp.float32),
            pltpu.SemaphoreType.DMA,
            pltpu.SemaphoreType.DMA,
            pltpu.SemaphoreType.DMA,
            pltpu.SemaphoreType.DMA,
            pltpu.SemaphoreType.DMA,
            pltpu.SemaphoreType.DMA,
        ],
    )
    def sc_kernel(items_hbm, users_hbm, hist_hbm, emb_item_hbm, emb_user_hbm,
                  it_out, ue_out, uh_out,
                  iidx, uidx, hidx, irows, urows, h0, h1, h2, h3, uh,
                  dbuf, sbuf,
                  sem_i, sem_u, sem_h0, sem_h1, sem_h2, sem_h3):
        wid = lax.axis_index("s") * NC + lax.axis_index("c")
        base = wid * b_per_w

        pltpu.sync_copy(items_hbm.at[pl.ds(base, b_per_w)], iidx)
        pltpu.sync_copy(users_hbm.at[pl.ds(base, b_per_w)], uidx)
        pltpu.sync_copy(hist_hbm.at[pl.ds(base, b_per_w)], hidx)

        item_cp = pltpu.make_async_copy(emb_item_hbm.at[iidx], irows, sem_i)
        item_cp.start()
        user_cp = pltpu.make_async_copy(emb_user_hbm.at[uidx], urows, sem_u)
        user_cp.start()
        user_cp.wait()

        hbufs = (h0, h1, h2, h3)
        hsems = (sem_h0, sem_h1, sem_h2, sem_h3)
        nbuf = len(hbufs)

        def hist_cp(b, k):
            return pltpu.make_async_copy(
                emb_user_hbm.at[hidx.at[b]], hbufs[k], hsems[k])

        for k in range(nbuf):
            hist_cp(k, k).start()

        iotav = lax.iota(jnp.int32, L) * (L + 1)
        zv = jnp.zeros((L,), jnp.float32)

        for r in range(H, HP):
            dbuf[pl.ds(r * (L + 1), L)] = zv

        def attend(b, hbuf):
            uch = [urows[b, pl.ds(c * L, L)] for c in range(NCH)]

            @plsc.parallel_loop(0, H, unroll=2)
            def _(h):
                hch = [hbuf[h, pl.ds(c * L, L)] for c in range(NCH)]
                dot = hch[0] * uch[0]
                for c in range(1, NCH):
                    dot = dot + hch[c] * uch[c]
                dbuf[pl.ds(h * (L + 1), L)] = dot

            @plsc.parallel_loop(0, HP // L)
            def _(g):
                gbase = g * (L * (L + 1))
                s = plsc.load_gather(dbuf, [iotav + gbase])
                for c in range(1, L):
                    s = s + plsc.load_gather(dbuf, [iotav + (gbase + c)])
                sbuf[pl.ds(g * L, L)] = s

            def p2(h, acc):
                sh = sbuf[pl.ds(h, L)][0]
                hch = [hbuf[h, pl.ds(c * L, L)] for c in range(NCH)]
                return tuple(acc[c] + sh * hch[c] for c in range(NCH))

            acc = plsc.parallel_loop(
                0, H, unroll=2, carry=tuple(zv for _ in range(NCH)))(p2)
            for c in range(NCH):
                uh[b, pl.ds(c * L, L)] = acc[c]

        def body(i, carry):
            for k in range(nbuf):
                b = nbuf * i + k
                hist_cp(b, k).wait()

                @pl.when(b + nbuf < b_per_w)
                def _():
                    hist_cp(b + nbuf, k).start()

                attend(b, hbufs[k])
            return carry

        lax.fori_loop(0, b_per_w // nbuf, body, 0)
        item_cp.wait()

        pltpu.sync_copy(irows, it_out.at[pl.ds(base, b_per_w)])
        pltpu.sync_copy(urows, ue_out.at[pl.ds(base, b_per_w)])
        pltpu.sync_copy(uh, uh_out.at[pl.ds(base, b_per_w)])

    return sc_kernel(items, users, history_users, emb_item, emb_user)


def _bn_dice(x, gamma, beta, alpha, n_feat):
    x = x * (gamma / jnp.sqrt(1.0 + 1e-5)) + beta
    avg = jnp.sum(x, axis=1, keepdims=True) * (1.0 / n_feat)
    cen = x - avg
    var = jnp.sum(cen * cen, axis=1, keepdims=True) + n_feat * 1e-3
    ps = jax.nn.sigmoid(cen / jnp.sqrt(var))
    return (ps + (1.0 - ps) * alpha) * x


def _tc_mlp(it, uh, ue, W1T, b1, g1, be1, a1, W2T, b2, g2, be2, a2, w3, b3):
    B = it.shape[0]
    F1 = W1T.shape[1]
    F2 = W2T.shape[1]
    BT = 1024 if B % 1024 == 0 else B
    grid = B // BT

    def body(it_ref, uh_ref, ue_ref, w1_ref, b1_ref, g1_ref, be1_ref, a1_ref,
             w2_ref, b2_ref, g2_ref, be2_ref, a2_ref, w3_ref, b3_ref, o_ref):
        h = (jnp.dot(it_ref[...], w1_ref[0:D, :],
                     preferred_element_type=jnp.float32)
             + jnp.dot(uh_ref[...], w1_ref[D:2 * D, :],
                       preferred_element_type=jnp.float32)
             + jnp.dot(ue_ref[...], w1_ref[2 * D:3 * D, :],
                       preferred_element_type=jnp.float32)
             + b1_ref[...])
        h = _bn_dice(h, g1_ref[...], be1_ref[...], a1_ref[0, 0], float(F1))
        h = jnp.dot(h, w2_ref[...], preferred_element_type=jnp.float32) \
            + b2_ref[...]
        h = _bn_dice(h, g2_ref[...], be2_ref[...], a2_ref[0, 0], float(F2))
        o_ref[...] = (jnp.sum(h * w3_ref[...], axis=1, keepdims=True)
                      + b3_ref[0, 0])

    row_spec = pl.BlockSpec((BT, D), lambda i: (i, 0))

    def rep(shape):
        return pl.BlockSpec(shape, lambda i: (0,) * len(shape))

    return pl.pallas_call(
        body,
        grid=(grid,),
        in_specs=[
            row_spec, row_spec, row_spec,
            rep((3 * D, F1)), rep((1, F1)), rep((1, F1)), rep((1, F1)),
            rep((1, 1)),
            rep((F1, F2)), rep((1, F2)), rep((1, F2)), rep((1, F2)),
            rep((1, 1)),
            rep((1, F2)), rep((1, 1)),
        ],
        out_specs=pl.BlockSpec((BT, 1), lambda i: (i, 0)),
        out_shape=jax.ShapeDtypeStruct((B, 1), jnp.float32),
    )(it, uh, ue, W1T, b1.reshape(1, F1), g1.reshape(1, F1),
      be1.reshape(1, F1), a1.reshape(1, 1), W2T, b2.reshape(1, F2),
      g2.reshape(1, F2), be2.reshape(1, F2), a2.reshape(1, 1),
      w3, b3.reshape(1, 1))


def kernel(items, users, history_users, emb_item, emb_user,
           W1, b1, g1, be1, a1, W2, b2, g2, be2, a2, W3, b3):
    B = items.shape[0]
    Bh = B // 2
    W1T, W2T = W1.T, W2.T
    outs = []
    for lo in (0, Bh):
        it, ue, uh = _sc_gather_attend(
            items[lo:lo + Bh].astype(jnp.int32),
            users[lo:lo + Bh].astype(jnp.int32),
            history_users[lo:lo + Bh].astype(jnp.int32),
            emb_item, emb_user)
        outs.append(_tc_mlp(it, uh, ue, W1T, b1, g1, be1, a1,
                            W2T, b2, g2, be2, a2, W3, b3))
    return jnp.concatenate(outs, axis=0)


# speedup vs baseline: 1.0299x; 1.0299x over previous
import functools

import jax
import jax.numpy as jnp
from jax import lax
from jax.experimental import pallas as pl
from jax.experimental.pallas import tpu as pltpu
from jax.experimental.pallas import tpu_sc as plsc

D = 128
H = 50
L = 16
NC = 2
NS = 16
NW = NC * NS
NCH = D // L
HP = 64


def _sc_gather_attend(items, users, history_users, emb_item, emb_user):
    B = items.shape[0]
    b_per_w = B // NW
    mesh = plsc.VectorSubcoreMesh(
        core_axis_name="c", subcore_axis_name="s",
        num_cores=NC, num_subcores=NS)

    @functools.partial(
        pl.kernel,
        mesh=mesh,
        compiler_params=pltpu.CompilerParams(needs_layout_passes=False),
        out_type=(
            jax.ShapeDtypeStruct((B, D), jnp.float32),
            jax.ShapeDtypeStruct((B, D), jnp.float32),
            jax.ShapeDtypeStruct((B, D), jnp.float32),
        ),
        scratch_types=[
            pltpu.VMEM((b_per_w,), jnp.int32),
            pltpu.VMEM((b_per_w,), jnp.int32),
            pltpu.VMEM((b_per_w, H), jnp.int32),
            pltpu.VMEM((b_per_w, D), jnp.float32),
            pltpu.VMEM((b_per_w, D), jnp.float32),
            pltpu.VMEM((H, D), jnp.float32),
            pltpu.VMEM((H, D), jnp.float32),
            pltpu.VMEM((H, D), jnp.float32),
            pltpu.VMEM((H, D), jnp.float32),
            pltpu.VMEM((b_per_w, D), jnp.float32),
            pltpu.VMEM((HP * (L + 1),), jnp.float32),
            pltpu.VMEM((HP + L,), jnp.float32),
            pltpu.SemaphoreType.DMA,
            pltpu.SemaphoreType.DMA,
            pltpu.SemaphoreType.DMA,
            pltpu.SemaphoreType.DMA,
            pltpu.SemaphoreType.DMA,
            pltpu.SemaphoreType.DMA,
        ],
    )
    def sc_kernel(items_hbm, users_hbm, hist_hbm, emb_item_hbm, emb_user_hbm,
                  it_out, ue_out, uh_out,
                  iidx, uidx, hidx, irows, urows, h0, h1, h2, h3, uh,
                  dbuf, sbuf,
                  sem_i, sem_u, sem_h0, sem_h1, sem_h2, sem_h3):
        wid = lax.axis_index("s") * NC + lax.axis_index("c")
        base = wid * b_per_w

        pltpu.sync_copy(items_hbm.at[pl.ds(base, b_per_w)], iidx)
        pltpu.sync_copy(users_hbm.at[pl.ds(base, b_per_w)], uidx)
        pltpu.sync_copy(hist_hbm.at[pl.ds(base, b_per_w)], hidx)

        item_cp = pltpu.make_async_copy(emb_item_hbm.at[iidx], irows, sem_i)
        item_cp.start()
        user_cp = pltpu.make_async_copy(emb_user_hbm.at[uidx], urows, sem_u)
        user_cp.start()
        user_cp.wait()

        hbufs = (h0, h1, h2, h3)
        hsems = (sem_h0, sem_h1, sem_h2, sem_h3)
        nbuf = len(hbufs)

        def hist_cp(b, k):
            return pltpu.make_async_copy(
                emb_user_hbm.at[hidx.at[b]], hbufs[k], hsems[k])

        for k in range(nbuf):
            hist_cp(k, k).start()

        iotav = lax.iota(jnp.int32, L) * (L + 1)
        zv = jnp.zeros((L,), jnp.float32)

        for r in range(H, HP):
            dbuf[pl.ds(r * (L + 1), L)] = zv

        def attend(b, hbuf):
            uch = [urows[b, pl.ds(c * L, L)] for c in range(NCH)]

            @plsc.parallel_loop(0, H, unroll=2)
            def _(h):
                hch = [hbuf[h, pl.ds(c * L, L)] for c in range(NCH)]
                dot = hch[0] * uch[0]
                for c in range(1, NCH):
                    dot = dot + hch[c] * uch[c]
                dbuf[pl.ds(h * (L + 1), L)] = dot

            @plsc.parallel_loop(0, HP // L)
            def _(g):
                gbase = g * (L * (L + 1))
                s = plsc.load_gather(dbuf, [iotav + gbase])
                for c in range(1, L):
                    s = s + plsc.load_gather(dbuf, [iotav + (gbase + c)])
                sbuf[pl.ds(g * L, L)] = s

            def p2(h, acc):
                sh = sbuf[pl.ds(h, L)][0]
                hch = [hbuf[h, pl.ds(c * L, L)] for c in range(NCH)]
                return tuple(acc[c] + sh * hch[c] for c in range(NCH))

            acc = plsc.parallel_loop(
                0, H, unroll=2, carry=tuple(zv for _ in range(NCH)))(p2)
            for c in range(NCH):
                uh[b, pl.ds(c * L, L)] = acc[c]

        def body(i, carry):
            for k in range(nbuf):
                b = nbuf * i + k
                hist_cp(b, k).wait()

                @pl.when(b + nbuf < b_per_w)
                def _():
                    hist_cp(b + nbuf, k).start()

                attend(b, hbufs[k])
            return carry

        lax.fori_loop(0, b_per_w // nbuf, body, 0)
        item_cp.wait()

        pltpu.sync_copy(irows, it_out.at[pl.ds(base, b_per_w)])
        pltpu.sync_copy(urows, ue_out.at[pl.ds(base, b_per_w)])
        pltpu.sync_copy(uh, uh_out.at[pl.ds(base, b_per_w)])

    return sc_kernel(items, users, history_users, emb_item, emb_user)


def _bn_dice(x, gamma, beta, alpha, n_feat):
    x = x * (gamma / jnp.sqrt(1.0 + 1e-5)) + beta
    avg = jnp.sum(x, axis=1, keepdims=True) * (1.0 / n_feat)
    cen = x - avg
    var = jnp.sum(cen * cen, axis=1, keepdims=True) + n_feat * 1e-3
    ps = jax.nn.sigmoid(cen / jnp.sqrt(var))
    return (ps + (1.0 - ps) * alpha) * x


def _tc_mlp(it, uh, ue, W1, b1, g1, be1, a1, W2, b2, g2, be2, a2, w3, b3):
    B = it.shape[0]
    F1 = W1.shape[0]
    F2 = W2.shape[0]
    dn = (((1,), (1,)), ((), ()))
    BT = 1024 if B % 1024 == 0 else B
    grid = B // BT

    def body(it_ref, uh_ref, ue_ref, w1_ref, b1_ref, g1_ref, be1_ref, a1_ref,
             w2_ref, b2_ref, g2_ref, be2_ref, a2_ref, w3_ref, b3_ref, o_ref):
        h = (lax.dot_general(it_ref[...], w1_ref[:, 0:D], dn,
                             preferred_element_type=jnp.float32)
             + lax.dot_general(uh_ref[...], w1_ref[:, D:2 * D], dn,
                               preferred_element_type=jnp.float32)
             + lax.dot_general(ue_ref[...], w1_ref[:, 2 * D:3 * D], dn,
                               preferred_element_type=jnp.float32)
             + b1_ref[...])
        h = _bn_dice(h, g1_ref[...], be1_ref[...], a1_ref[0, 0], float(F1))
        h = lax.dot_general(h, w2_ref[...], dn,
                            preferred_element_type=jnp.float32) + b2_ref[...]
        h = _bn_dice(h, g2_ref[...], be2_ref[...], a2_ref[0, 0], float(F2))
        o_ref[...] = (jnp.sum(h * w3_ref[...], axis=1, keepdims=True)
                      + b3_ref[0, 0])

    row_spec = pl.BlockSpec((BT, D), lambda i: (i, 0))

    def rep(shape):
        return pl.BlockSpec(shape, lambda i: (0,) * len(shape))

    return pl.pallas_call(
        body,
        grid=(grid,),
        in_specs=[
            row_spec, row_spec, row_spec,
            rep((F1, 3 * D)), rep((1, F1)), rep((1, F1)), rep((1, F1)),
            rep((1, 1)),
            rep((F2, F1)), rep((1, F2)), rep((1, F2)), rep((1, F2)),
            rep((1, 1)),
            rep((1, F2)), rep((1, 1)),
        ],
        out_specs=pl.BlockSpec((BT, 1), lambda i: (i, 0)),
        out_shape=jax.ShapeDtypeStruct((B, 1), jnp.float32),
    )(it, uh, ue, W1, b1.reshape(1, F1), g1.reshape(1, F1),
      be1.reshape(1, F1), a1.reshape(1, 1), W2, b2.reshape(1, F2),
      g2.reshape(1, F2), be2.reshape(1, F2), a2.reshape(1, 1),
      w3, b3.reshape(1, 1))


def kernel(items, users, history_users, emb_item, emb_user,
           W1, b1, g1, be1, a1, W2, b2, g2, be2, a2, W3, b3):
    it, ue, uh = _sc_gather_attend(
        items.astype(jnp.int32), users.astype(jnp.int32),
        history_users.astype(jnp.int32), emb_item, emb_user)
    return _tc_mlp(it, uh, ue, W1, b1, g1, be1, a1,
                   W2, b2, g2, be2, a2, W3, b3)


# speedup vs baseline: 1.0388x; 1.0086x over previous
import functools

import jax
import jax.numpy as jnp
from jax import lax
from jax.experimental import pallas as pl
from jax.experimental.pallas import tpu as pltpu
from jax.experimental.pallas import tpu_sc as plsc

D = 128
H = 50
L = 16
NC = 2
NS = 16
NW = NC * NS
NCH = D // L
HP = 64


def _sc_gather_attend(items, users, history_users, emb_item, emb_user):
    B = items.shape[0]
    b_per_w = B // NW
    mesh = plsc.VectorSubcoreMesh(
        core_axis_name="c", subcore_axis_name="s",
        num_cores=NC, num_subcores=NS)

    @functools.partial(
        pl.kernel,
        mesh=mesh,
        compiler_params=pltpu.CompilerParams(needs_layout_passes=False),
        out_type=(
            jax.ShapeDtypeStruct((B, D), jnp.float32),
            jax.ShapeDtypeStruct((B, D), jnp.float32),
            jax.ShapeDtypeStruct((B, D), jnp.float32),
        ),
        scratch_types=[
            pltpu.VMEM((b_per_w,), jnp.int32),
            pltpu.VMEM((b_per_w,), jnp.int32),
            pltpu.VMEM((b_per_w, H), jnp.int32),
            pltpu.VMEM((b_per_w, D), jnp.float32),
            pltpu.VMEM((b_per_w, D), jnp.float32),
            pltpu.VMEM((H, D), jnp.float32),
            pltpu.VMEM((H, D), jnp.float32),
            pltpu.VMEM((H, D), jnp.float32),
            pltpu.VMEM((H, D), jnp.float32),
            pltpu.VMEM((b_per_w, D), jnp.float32),
            pltpu.VMEM((HP * (L + 1),), jnp.float32),
            pltpu.VMEM((HP + L,), jnp.float32),
            pltpu.SemaphoreType.DMA,
            pltpu.SemaphoreType.DMA,
            pltpu.SemaphoreType.DMA,
            pltpu.SemaphoreType.DMA,
            pltpu.SemaphoreType.DMA,
            pltpu.SemaphoreType.DMA,
        ],
    )
    def sc_kernel(items_hbm, users_hbm, hist_hbm, emb_item_hbm, emb_user_hbm,
                  it_out, ue_out, uh_out,
                  iidx, uidx, hidx, irows, urows, h0, h1, h2, h3, uh,
                  dbuf, sbuf,
                  sem_i, sem_u, sem_h0, sem_h1, sem_h2, sem_h3):
        wid = lax.axis_index("s") * NC + lax.axis_index("c")
        base = wid * b_per_w

        pltpu.sync_copy(items_hbm.at[pl.ds(base, b_per_w)], iidx)
        pltpu.sync_copy(users_hbm.at[pl.ds(base, b_per_w)], uidx)
        pltpu.sync_copy(hist_hbm.at[pl.ds(base, b_per_w)], hidx)

        item_cp = pltpu.make_async_copy(emb_item_hbm.at[iidx], irows, sem_i)
        item_cp.start()
        user_cp = pltpu.make_async_copy(emb_user_hbm.at[uidx], urows, sem_u)
        user_cp.start()
        user_cp.wait()

        hbufs = (h0, h1, h2, h3)
        hsems = (sem_h0, sem_h1, sem_h2, sem_h3)
        nbuf = len(hbufs)

        def hist_cp(b, k):
            return pltpu.make_async_copy(
                emb_user_hbm.at[hidx.at[b]], hbufs[k], hsems[k])

        for k in range(nbuf):
            hist_cp(k, k).start()

        iotav = lax.iota(jnp.int32, L) * (L + 1)
        zv = jnp.zeros((L,), jnp.float32)

        for r in range(H, HP):
            dbuf[pl.ds(r * (L + 1), L)] = zv

        def attend(b, hbuf):
            uch = [urows[b, pl.ds(c * L, L)] for c in range(NCH)]

            @plsc.parallel_loop(0, H, unroll=2)
            def _(h):
                hch = [hbuf[h, pl.ds(c * L, L)] for c in range(NCH)]
                m = [hch[c] * uch[c] for c in range(NCH)]
                while len(m) > 1:
                    m = [m[i] + m[i + 1] for i in range(0, len(m), 2)]
                dbuf[pl.ds(h * (L + 1), L)] = m[0]

            @plsc.parallel_loop(0, HP // L)
            def _(g):
                gbase = g * (L * (L + 1))
                m = [plsc.load_gather(dbuf, [iotav + (gbase + c)])
                     for c in range(L)]
                while len(m) > 1:
                    m = [m[i] + m[i + 1] for i in range(0, len(m), 2)]
                sbuf[pl.ds(g * L, L)] = m[0]

            def p2(h, acc):
                sh = sbuf[pl.ds(h, L)][0]
                hch = [hbuf[h, pl.ds(c * L, L)] for c in range(NCH)]
                return tuple(acc[c] + sh * hch[c] for c in range(NCH))

            acc = plsc.parallel_loop(
                0, H, unroll=2, carry=tuple(zv for _ in range(NCH)))(p2)
            for c in range(NCH):
                uh[b, pl.ds(c * L, L)] = acc[c]

        def body(i, carry):
            for k in range(nbuf):
                b = nbuf * i + k
                hist_cp(b, k).wait()

                @pl.when(b + nbuf < b_per_w)
                def _():
                    hist_cp(b + nbuf, k).start()

                attend(b, hbufs[k])
            return carry

        lax.fori_loop(0, b_per_w // nbuf, body, 0)
        item_cp.wait()

        pltpu.sync_copy(irows, it_out.at[pl.ds(base, b_per_w)])
        pltpu.sync_copy(urows, ue_out.at[pl.ds(base, b_per_w)])
        pltpu.sync_copy(uh, uh_out.at[pl.ds(base, b_per_w)])

    return sc_kernel(items, users, history_users, emb_item, emb_user)


def _bn_dice(x, gamma, beta, alpha, n_feat):
    x = x * (gamma / jnp.sqrt(1.0 + 1e-5)) + beta
    avg = jnp.sum(x, axis=1, keepdims=True) * (1.0 / n_feat)
    cen = x - avg
    var = jnp.sum(cen * cen, axis=1, keepdims=True) + n_feat * 1e-3
    ps = jax.nn.sigmoid(cen / jnp.sqrt(var))
    return (ps + (1.0 - ps) * alpha) * x


def _tc_mlp(it, uh, ue, W1, b1, g1, be1, a1, W2, b2, g2, be2, a2, w3, b3):
    B = it.shape[0]
    F1 = W1.shape[0]
    F2 = W2.shape[0]
    dn = (((1,), (1,)), ((), ()))
    BT = 1024 if B % 1024 == 0 else B
    grid = B // BT

    def body(it_ref, uh_ref, ue_ref, w1_ref, b1_ref, g1_ref, be1_ref, a1_ref,
             w2_ref, b2_ref, g2_ref, be2_ref, a2_ref, w3_ref, b3_ref, o_ref):
        h = (lax.dot_general(it_ref[...], w1_ref[:, 0:D], dn,
                             preferred_element_type=jnp.float32)
             + lax.dot_general(uh_ref[...], w1_ref[:, D:2 * D], dn,
                               preferred_element_type=jnp.float32)
             + lax.dot_general(ue_ref[...], w1_ref[:, 2 * D:3 * D], dn,
                               preferred_element_type=jnp.float32)
             + b1_ref[...])
        h = _bn_dice(h, g1_ref[...], be1_ref[...], a1_ref[0, 0], float(F1))
        h = lax.dot_general(h, w2_ref[...], dn,
                            preferred_element_type=jnp.float32) + b2_ref[...]
        h = _bn_dice(h, g2_ref[...], be2_ref[...], a2_ref[0, 0], float(F2))
        o_ref[...] = (jnp.sum(h * w3_ref[...], axis=1, keepdims=True)
                      + b3_ref[0, 0])

    row_spec = pl.BlockSpec((BT, D), lambda i: (i, 0))

    def rep(shape):
        return pl.BlockSpec(shape, lambda i: (0,) * len(shape))

    return pl.pallas_call(
        body,
        grid=(grid,),
        in_specs=[
            row_spec, row_spec, row_spec,
            rep((F1, 3 * D)), rep((1, F1)), rep((1, F1)), rep((1, F1)),
            rep((1, 1)),
            rep((F2, F1)), rep((1, F2)), rep((1, F2)), rep((1, F2)),
            rep((1, 1)),
            rep((1, F2)), rep((1, 1)),
        ],
        out_specs=pl.BlockSpec((BT, 1), lambda i: (i, 0)),
        out_shape=jax.ShapeDtypeStruct((B, 1), jnp.float32),
    )(it, uh, ue, W1, b1.reshape(1, F1), g1.reshape(1, F1),
      be1.reshape(1, F1), a1.reshape(1, 1), W2, b2.reshape(1, F2),
      g2.reshape(1, F2), be2.reshape(1, F2), a2.reshape(1, 1),
      w3, b3.reshape(1, 1))


def kernel(items, users, history_users, emb_item, emb_user,
           W1, b1, g1, be1, a1, W2, b2, g2, be2, a2, W3, b3):
    it, ue, uh = _sc_gather_attend(
        items.astype(jnp.int32), users.astype(jnp.int32),
        history_users.astype(jnp.int32), emb_item, emb_user)
    return _tc_mlp(it, uh, ue, W1, b1, g1, be1, a1,
                   W2, b2, g2, be2, a2, W3, b3)


# speedup vs baseline: 1.0436x; 1.0047x over previous
import functools

import jax
import jax.numpy as jnp
from jax import lax
from jax.experimental import pallas as pl
from jax.experimental.pallas import tpu as pltpu
from jax.experimental.pallas import tpu_sc as plsc

D = 128
H = 50
L = 16
NC = 2
NS = 16
NW = NC * NS
NCH = D // L
HP = 64


def _sc_gather_attend(items, users, history_users, emb_item, emb_user):
    B = items.shape[0]
    b_per_w = B // NW
    mesh = plsc.VectorSubcoreMesh(
        core_axis_name="c", subcore_axis_name="s",
        num_cores=NC, num_subcores=NS)

    @functools.partial(
        pl.kernel,
        mesh=mesh,
        compiler_params=pltpu.CompilerParams(needs_layout_passes=False),
        out_type=(
            jax.ShapeDtypeStruct((B, D), jnp.float32),
            jax.ShapeDtypeStruct((B, D), jnp.float32),
            jax.ShapeDtypeStruct((B, D), jnp.float32),
        ),
        scratch_types=[
            pltpu.VMEM((b_per_w,), jnp.int32),
            pltpu.VMEM((b_per_w,), jnp.int32),
            pltpu.VMEM((b_per_w, H), jnp.int32),
            pltpu.VMEM((b_per_w, D), jnp.float32),
            pltpu.VMEM((b_per_w, D), jnp.float32),
            pltpu.VMEM((H, D), jnp.float32),
            pltpu.VMEM((H, D), jnp.float32),
            pltpu.VMEM((H, D), jnp.float32),
            pltpu.VMEM((H, D), jnp.float32),
            pltpu.VMEM((b_per_w, D), jnp.float32),
            pltpu.VMEM((HP * (L + 1),), jnp.float32),
            pltpu.VMEM((HP + L,), jnp.float32),
            pltpu.SemaphoreType.DMA,
            pltpu.SemaphoreType.DMA,
            pltpu.SemaphoreType.DMA,
            pltpu.SemaphoreType.DMA,
            pltpu.SemaphoreType.DMA,
            pltpu.SemaphoreType.DMA,
        ],
    )
    def sc_kernel(items_hbm, users_hbm, hist_hbm, emb_item_hbm, emb_user_hbm,
                  it_out, ue_out, uh_out,
                  iidx, uidx, hidx, irows, urows, h0, h1, h2, h3, uh,
                  dbuf, sbuf,
                  sem_i, sem_u, sem_h0, sem_h1, sem_h2, sem_h3):
        wid = lax.axis_index("s") * NC + lax.axis_index("c")
        base = wid * b_per_w

        pltpu.sync_copy(items_hbm.at[pl.ds(base, b_per_w)], iidx)
        pltpu.sync_copy(users_hbm.at[pl.ds(base, b_per_w)], uidx)
        pltpu.sync_copy(hist_hbm.at[pl.ds(base, b_per_w)], hidx)

        item_cp = pltpu.make_async_copy(emb_item_hbm.at[iidx], irows, sem_i)
        item_cp.start()
        user_cp = pltpu.make_async_copy(emb_user_hbm.at[uidx], urows, sem_u)
        user_cp.start()
        user_cp.wait()

        hbufs = (h0, h1, h2, h3)
        hsems = (sem_h0, sem_h1, sem_h2, sem_h3)
        nbuf = len(hbufs)

        def hist_cp(b, k):
            return pltpu.make_async_copy(
                emb_user_hbm.at[hidx.at[b]], hbufs[k], hsems[k])

        for k in range(nbuf):
            hist_cp(k, k).start()

        iotav = lax.iota(jnp.int32, L) * (L + 1)
        zv = jnp.zeros((L,), jnp.float32)

        for r in range(H, HP):
            dbuf[pl.ds(r * (L + 1), L)] = zv

        def attend(b, hbuf):
            uch = [urows[b, pl.ds(c * L, L)] for c in range(NCH)]

            @plsc.parallel_loop(0, H, unroll=5)
            def _(h):
                hch = [hbuf[h, pl.ds(c * L, L)] for c in range(NCH)]
                m = [hch[c] * uch[c] for c in range(NCH)]
                while len(m) > 1:
                    m = [m[i] + m[i + 1] for i in range(0, len(m), 2)]
                dbuf[pl.ds(h * (L + 1), L)] = m[0]

            @plsc.parallel_loop(0, HP // L)
            def _(g):
                gbase = g * (L * (L + 1))
                m = [plsc.load_gather(dbuf, [iotav + (gbase + c)])
                     for c in range(L)]
                while len(m) > 1:
                    m = [m[i] + m[i + 1] for i in range(0, len(m), 2)]
                sbuf[pl.ds(g * L, L)] = m[0]

            def p2(h, acc):
                sh = sbuf[pl.ds(h, L)][0]
                hch = [hbuf[h, pl.ds(c * L, L)] for c in range(NCH)]
                return tuple(acc[c] + sh * hch[c] for c in range(NCH))

            acc = plsc.parallel_loop(
                0, H, unroll=5, carry=tuple(zv for _ in range(NCH)))(p2)
            for c in range(NCH):
                uh[b, pl.ds(c * L, L)] = acc[c]

        def body(i, carry):
            for k in range(nbuf):
                b = nbuf * i + k
                hist_cp(b, k).wait()

                @pl.when(b + nbuf < b_per_w)
                def _():
                    hist_cp(b + nbuf, k).start()

                attend(b, hbufs[k])
            return carry

        lax.fori_loop(0, b_per_w // nbuf, body, 0)
        item_cp.wait()

        pltpu.sync_copy(irows, it_out.at[pl.ds(base, b_per_w)])
        pltpu.sync_copy(urows, ue_out.at[pl.ds(base, b_per_w)])
        pltpu.sync_copy(uh, uh_out.at[pl.ds(base, b_per_w)])

    return sc_kernel(items, users, history_users, emb_item, emb_user)


def _bn_dice(x, gamma, beta, alpha, n_feat):
    x = x * (gamma / jnp.sqrt(1.0 + 1e-5)) + beta
    avg = jnp.sum(x, axis=1, keepdims=True) * (1.0 / n_feat)
    cen = x - avg
    var = jnp.sum(cen * cen, axis=1, keepdims=True) + n_feat * 1e-3
    ps = jax.nn.sigmoid(cen / jnp.sqrt(var))
    return (ps + (1.0 - ps) * alpha) * x


def _tc_mlp(it, uh, ue, W1, b1, g1, be1, a1, W2, b2, g2, be2, a2, w3, b3):
    B = it.shape[0]
    F1 = W1.shape[0]
    F2 = W2.shape[0]
    dn = (((1,), (1,)), ((), ()))
    BT = 1024 if B % 1024 == 0 else B
    grid = B // BT

    def body(it_ref, uh_ref, ue_ref, w1_ref, b1_ref, g1_ref, be1_ref, a1_ref,
             w2_ref, b2_ref, g2_ref, be2_ref, a2_ref, w3_ref, b3_ref, o_ref):
        h = (lax.dot_general(it_ref[...], w1_ref[:, 0:D], dn,
                             preferred_element_type=jnp.float32)
             + lax.dot_general(uh_ref[...], w1_ref[:, D:2 * D], dn,
                               preferred_element_type=jnp.float32)
             + lax.dot_general(ue_ref[...], w1_ref[:, 2 * D:3 * D], dn,
                               preferred_element_type=jnp.float32)
             + b1_ref[...])
        h = _bn_dice(h, g1_ref[...], be1_ref[...], a1_ref[0, 0], float(F1))
        h = lax.dot_general(h, w2_ref[...], dn,
                            preferred_element_type=jnp.float32) + b2_ref[...]
        h = _bn_dice(h, g2_ref[...], be2_ref[...], a2_ref[0, 0], float(F2))
        o_ref[...] = (jnp.sum(h * w3_ref[...], axis=1, keepdims=True)
                      + b3_ref[0, 0])

    row_spec = pl.BlockSpec((BT, D), lambda i: (i, 0))

    def rep(shape):
        return pl.BlockSpec(shape, lambda i: (0,) * len(shape))

    return pl.pallas_call(
        body,
        grid=(grid,),
        in_specs=[
            row_spec, row_spec, row_spec,
            rep((F1, 3 * D)), rep((1, F1)), rep((1, F1)), rep((1, F1)),
            rep((1, 1)),
            rep((F2, F1)), rep((1, F2)), rep((1, F2)), rep((1, F2)),
            rep((1, 1)),
            rep((1, F2)), rep((1, 1)),
        ],
        out_specs=pl.BlockSpec((BT, 1), lambda i: (i, 0)),
        out_shape=jax.ShapeDtypeStruct((B, 1), jnp.float32),
    )(it, uh, ue, W1, b1.reshape(1, F1), g1.reshape(1, F1),
      be1.reshape(1, F1), a1.reshape(1, 1), W2, b2.reshape(1, F2),
      g2.reshape(1, F2), be2.reshape(1, F2), a2.reshape(1, 1),
      w3, b3.reshape(1, 1))


def kernel(items, users, history_users, emb_item, emb_user,
           W1, b1, g1, be1, a1, W2, b2, g2, be2, a2, W3, b3):
    it, ue, uh = _sc_gather_attend(
        items.astype(jnp.int32), users.astype(jnp.int32),
        history_users.astype(jnp.int32), emb_item, emb_user)
    return _tc_mlp(it, uh, ue, W1, b1, g1, be1, a1,
                   W2, b2, g2, be2, a2, W3, b3)
